# Initial kernel scaffold; baseline (speedup 1.0000x reference)
import jax
import jax.numpy as jnp
from jax.experimental import pallas as pl


def kernel(x, labels, scopes, W, b):
    raise NotImplementedError("write your pallas kernel here")


# speedup vs baseline: 5.3947x; 5.3947x over previous
import jax
import jax.numpy as jnp
import numpy as np
from jax.experimental import pallas as pl
from jax.experimental.pallas import tpu as pltpu


def _att_bag_kernel(bag_labels_ref, x_ref, w_ref, b_ref, repre_ref, logits_ref):
    i = pl.program_id(0)
    lab = bag_labels_ref[i]
    C = w_ref.shape[0]
    H = w_ref.shape[1]
    onehot = (jax.lax.broadcasted_iota(jnp.int32, (1, C), 1) == lab).astype(
        jnp.float32
    )
    w = jax.lax.dot_general(
        onehot, w_ref[...], (((1,), (0,)), ((), ())),
        preferred_element_type=jnp.float32,
    )

    x = x_ref[...]
    logit = jax.lax.dot_general(
        x, w, (((1,), (1,)), ((), ())), preferred_element_type=jnp.float32
    )
    m = jnp.max(logit)
    p = jnp.exp(logit - m)
    s = jnp.sum(p)
    acc = jax.lax.dot_general(
        p, x, (((0,), (0,)), ((), ())), preferred_element_type=jnp.float32
    )
    repre = acc * (1.0 / s)
    repre_ref[...] = repre.reshape(1, 1, H)
    row = jax.lax.dot_general(
        repre, w_ref[...], (((1,), (1,)), ((), ())),
        preferred_element_type=jnp.float32,
    ) + b_ref[...]
    logits_ref[...] = row.reshape(1, 1, C)


def kernel(x, labels, scopes, W, b):
    N, H = x.shape
    C = W.shape[0]
    B = scopes.shape[0]
    L = N // B

    starts = jnp.asarray(scopes)[:, 0].astype(jnp.int32)
    bag_labels = jnp.take(labels, starts, axis=0).astype(jnp.int32)
    b2 = b.reshape(1, C)

    grid_spec = pltpu.PrefetchScalarGridSpec(
        num_scalar_prefetch=1,
        grid=(B,),
        in_specs=[
            pl.BlockSpec((L, H), lambda i, *_: (i, 0)),
            pl.BlockSpec((C, H), lambda i, *_: (0, 0)),
            pl.BlockSpec((1, C), lambda i, *_: (0, 0)),
        ],
        out_specs=[
            pl.BlockSpec((1, 1, H), lambda i, *_: (i, 0, 0)),
            pl.BlockSpec((1, 1, C), lambda i, *_: (i, 0, 0)),
        ],
    )
    repre3, logits3 = pl.pallas_call(
        _att_bag_kernel,
        grid_spec=grid_spec,
        out_shape=[
            jax.ShapeDtypeStruct((B, 1, H), jnp.float32),
            jax.ShapeDtypeStruct((B, 1, C), jnp.float32),
        ],
    )(bag_labels, x, W, b2)
    return (repre3.reshape(B, H), logits3.reshape(B, C))
